# baseline (speedup 1.0000x reference)
import functools

import jax
import jax.numpy as jnp
from jax import lax
from jax.experimental import pallas as pl
from jax.experimental.pallas import tpu as pltpu
from jax.experimental.pallas import tpu_sc as plsc

N_NODES = 10000
D_FEAT = 128
N_EDGES = 320000

NC = 2
NS = 16
NW = NC * NS
CHUNK = 128
CPT = 80
E_PAD = NW * CPT * CHUNK
N_PAD = 10112
ROWS_PER_TILE = N_PAD // NS

_PARTS = ((0, 128), (128, 128), (256, 128), (384, 128), (512, 120))

IDX_STAGE = 8


def _mesh():
    return plsc.VectorSubcoreMesh(
        core_axis_name="c", subcore_axis_name="s", num_cores=NC)


CHUNK_G = 64
N_CHUNKS_G = E_PAD // CHUNK_G
CPT_FAST = 272
CPT_SLOW = (N_CHUNKS_G - NS * CPT_FAST) // NS
_PARTS_G = tuple((i * 64, 64) for i in range(9)) + ((576, 56),)


@functools.lru_cache(maxsize=None)
def _make_sc_pass():
    @functools.partial(
        pl.kernel,
        mesh=_mesh(),
        out_type=[jax.ShapeDtypeStruct((NC, N_PAD, D_FEAT), jnp.float32)],
        scratch_types=[
            pltpu.VMEM((IDX_STAGE, CHUNK_G), jnp.int32),
            pltpu.VMEM((IDX_STAGE, CHUNK_G), jnp.int32),
            pltpu.VMEM((CHUNK_G, D_FEAT), jnp.float32),
            pltpu.VMEM((CHUNK_G, D_FEAT), jnp.float32),
            pltpu.VMEM((CHUNK_G, D_FEAT), jnp.float32),
            pltpu.VMEM_SHARED((N_PAD, D_FEAT), jnp.float32),
            pltpu.SemaphoreType.DMA,
            pltpu.SemaphoreType.DMA,
            pltpu.SemaphoreType.DMA,
        ],
    )
    def sc_pass(x_hbm, src_hbm, dst_hbm, acc_out,
                src_v, dst_v, rows0, rows1, rows2, acc_sh, sem0, sem1, sem2):
        cid = lax.axis_index("c")
        sid = lax.axis_index("s")
        rows = (rows0, rows1, rows2)
        sems = (sem0, sem1, sem2)

        base = jnp.where(cid == 0, sid * CPT_FAST,
                         NS * CPT_FAST + sid * CPT_SLOW)
        n_gather = jnp.where(cid == 0, CPT_FAST // IDX_STAGE,
                             CPT_SLOW // IDX_STAGE)

        zrow = jnp.zeros((16,), jnp.float32)

        def fill_rows0(val):
            def body(i, carry):
                for j in range(D_FEAT // 16):
                    rows0[i, pl.ds(j * 16, 16)] = val
                return carry
            return body

        lax.fori_loop(0, CHUNK_G, fill_rows0(zrow), 0)

        r0 = sid * ROWS_PER_TILE
        for off, sz in _PARTS_G:
            pltpu.sync_copy(rows0.at[pl.ds(0, sz)],
                            acc_sh.at[pl.ds(r0 + off, sz)])

        plsc.subcore_barrier()

        def gather_body(s, carry):
            s8 = pl.multiple_of(base + s * IDX_STAGE, IDX_STAGE)
            pltpu.sync_copy(src_hbm.at[pl.ds(s8, IDX_STAGE)], src_v)
            pltpu.sync_copy(dst_hbm.at[pl.ds(s8, IDX_STAGE)], dst_v)
            handles = [None] * IDX_STAGE
            for j in range(2):
                handles[j] = pltpu.async_copy(
                    x_hbm.at[src_v.at[j]], rows[j], sems[j])
            for j in range(IDX_STAGE):
                if j + 2 < IDX_STAGE:
                    b = (j + 2) % 3
                    handles[j + 2] = pltpu.async_copy(
                        x_hbm.at[src_v.at[j + 2]], rows[b], sems[b])
                handles[j].wait()
                pltpu.sync_copy(rows[j % 3], acc_sh.at[dst_v.at[j]], add=True)
            return carry

        lax.fori_loop(0, n_gather, gather_body, 0)

        plsc.subcore_barrier()

        for off, sz in _PARTS_G:
            pltpu.sync_copy(acc_sh.at[pl.ds(r0 + off, sz)],
                            rows1.at[pl.ds(0, sz)])
            pltpu.sync_copy(rows1.at[pl.ds(0, sz)],
                            acc_out.at[cid, pl.ds(r0 + off, sz)])

    return sc_pass


@functools.lru_cache(maxsize=None)
def _make_deg_pass():
    @functools.partial(
        pl.kernel,
        mesh=_mesh(),
        out_type=[jax.ShapeDtypeStruct((NC, N_PAD, D_FEAT), jnp.float32)],
        scratch_types=[
            pltpu.VMEM((IDX_STAGE, CHUNK), jnp.int32),
            pltpu.VMEM((CHUNK, D_FEAT), jnp.float32),
            pltpu.VMEM_SHARED((N_PAD, D_FEAT), jnp.float32),
        ],
    )
    def deg_pass(dst_hbm, deg_out, dst_v, ones_v, deg_sh):
        cid = lax.axis_index("c")
        sid = lax.axis_index("s")
        wid = sid * NC + cid

        zrow = jnp.zeros((16,), jnp.float32)
        orow = jnp.full((16,), 1.0, jnp.float32)

        def fill_body(val):
            def body(i, carry):
                for j in range(D_FEAT // 16):
                    ones_v[i, pl.ds(j * 16, 16)] = val
                return carry
            return body

        lax.fori_loop(0, CHUNK, fill_body(zrow), 0)

        r0 = sid * ROWS_PER_TILE
        for off, sz in _PARTS:
            pltpu.sync_copy(ones_v.at[pl.ds(0, sz)],
                            deg_sh.at[pl.ds(r0 + off, sz)])

        lax.fori_loop(0, CHUNK, fill_body(orow), 0)

        plsc.subcore_barrier()

        def stage_body(s, carry):
            s8 = pl.multiple_of(s * IDX_STAGE, IDX_STAGE)
            pltpu.sync_copy(dst_hbm.at[wid, pl.ds(s8, IDX_STAGE)], dst_v)
            for j in range(IDX_STAGE):
                pltpu.sync_copy(ones_v, deg_sh.at[dst_v.at[j]], add=True)
            return carry

        lax.fori_loop(0, CPT // IDX_STAGE, stage_body, 0)

        plsc.subcore_barrier()

        for off, sz in _PARTS:
            pltpu.sync_copy(deg_sh.at[pl.ds(r0 + off, sz)],
                            ones_v.at[pl.ds(0, sz)])
            pltpu.sync_copy(ones_v.at[pl.ds(0, sz)],
                            deg_out.at[cid, pl.ds(r0 + off, sz)])

    return deg_pass


_TC_BLOCK = 400


def _tc_body(final, p_ref, deg_ref, x_ref, wl_ref, b_ref, wr_ref, o_ref):
    summed = p_ref[0] + p_ref[1]
    deg = deg_ref[0, :, 0:1] + deg_ref[1, :, 0:1]
    agg = summed / jnp.maximum(deg, 1.0)
    out = (jnp.dot(agg, wl_ref[...], precision=lax.Precision.HIGHEST)
           + b_ref[...]
           + jnp.dot(x_ref[...], wr_ref[...], precision=lax.Precision.HIGHEST))
    norm = jnp.sqrt(jnp.sum(out * out, axis=-1, keepdims=True))
    out = out / jnp.maximum(norm, 1e-12)
    out = jnp.maximum(out, 0.0)
    if final:
        m = jnp.max(out, axis=-1, keepdims=True)
        s = out - m
        lse = jnp.log(jnp.sum(jnp.exp(s), axis=-1, keepdims=True))
        out = s - lse
    o_ref[...] = out


def _tc_layer(p, degp, x_in, wl_t, b_row, wr_t, final, interpret=False):
    n = x_in.shape[0]
    return pl.pallas_call(
        functools.partial(_tc_body, final),
        grid=(n // _TC_BLOCK,),
        in_specs=[
            pl.BlockSpec((NC, _TC_BLOCK, D_FEAT), lambda i: (0, i, 0)),
            pl.BlockSpec((NC, _TC_BLOCK, D_FEAT), lambda i: (0, i, 0)),
            pl.BlockSpec((_TC_BLOCK, D_FEAT), lambda i: (i, 0)),
            pl.BlockSpec((D_FEAT, D_FEAT), lambda i: (0, 0)),
            pl.BlockSpec((1, D_FEAT), lambda i: (0, 0)),
            pl.BlockSpec((D_FEAT, D_FEAT), lambda i: (0, 0)),
        ],
        out_specs=pl.BlockSpec((_TC_BLOCK, D_FEAT), lambda i: (i, 0)),
        out_shape=jax.ShapeDtypeStruct((n, D_FEAT), jnp.float32),
        interpret=interpret,
    )(p, degp, x_in, wl_t, b_row, wr_t)


def kernel(x, edge_index, W_l1, b_l1, W_r1, W_l2, b_l2, W_r2):
    src = edge_index[0].astype(jnp.int32)
    dst = edge_index[1].astype(jnp.int32)
    pad = E_PAD - N_EDGES
    src_p = jnp.concatenate([src, jnp.zeros((pad,), jnp.int32)])
    dst_p = jnp.concatenate([dst, jnp.full((pad,), N_NODES, jnp.int32)])
    src_g = src_p.reshape(N_CHUNKS_G, CHUNK_G)
    dst_g = dst_p.reshape(N_CHUNKS_G, CHUNK_G)
    dst_d = dst_p.reshape(NW, CPT, CHUNK)

    d1 = _make_deg_pass()(dst_d)[0]
    p1 = _make_sc_pass()(x, src_g, dst_g)[0]
    h1 = _tc_layer(p1, d1, x, W_l1.T, b_l1.reshape(1, -1), W_r1.T, final=False)
    p2 = _make_sc_pass()(h1, src_g, dst_g)[0]
    out = _tc_layer(p2, d1, h1, W_l2.T, b_l2.reshape(1, -1), W_r2.T, final=True)
    return out


# speedup vs baseline: 1.0353x; 1.0353x over previous
import functools

import jax
import jax.numpy as jnp
from jax import lax
from jax.experimental import pallas as pl
from jax.experimental.pallas import tpu as pltpu
from jax.experimental.pallas import tpu_sc as plsc

N_NODES = 10000
D_FEAT = 128
N_EDGES = 320000

NC = 2
NS = 16
NW = NC * NS
CHUNK = 128
CPT = 80
E_PAD = NW * CPT * CHUNK
N_PAD = 10112
ROWS_PER_TILE = N_PAD // NS

_PARTS = ((0, 128), (128, 128), (256, 128), (384, 128), (512, 120))

IDX_STAGE = 8


def _mesh():
    return plsc.VectorSubcoreMesh(
        core_axis_name="c", subcore_axis_name="s", num_cores=NC)


CHUNK_G = 64
N_CHUNKS_G = E_PAD // CHUNK_G
CPT_FAST = 288
CPT_SLOW = (N_CHUNKS_G - NS * CPT_FAST) // NS
_PARTS_G = tuple((i * 64, 64) for i in range(9)) + ((576, 56),)


@functools.lru_cache(maxsize=None)
def _make_sc_pass():
    @functools.partial(
        pl.kernel,
        mesh=_mesh(),
        out_type=[jax.ShapeDtypeStruct((NC, N_PAD, D_FEAT), jnp.float32)],
        scratch_types=[
            pltpu.VMEM((IDX_STAGE, CHUNK_G), jnp.int32),
            pltpu.VMEM((IDX_STAGE, CHUNK_G), jnp.int32),
            pltpu.VMEM((CHUNK_G, D_FEAT), jnp.float32),
            pltpu.VMEM((CHUNK_G, D_FEAT), jnp.float32),
            pltpu.VMEM((CHUNK_G, D_FEAT), jnp.float32),
            pltpu.VMEM_SHARED((N_PAD, D_FEAT), jnp.float32),
            pltpu.SemaphoreType.DMA,
            pltpu.SemaphoreType.DMA,
            pltpu.SemaphoreType.DMA,
        ],
    )
    def sc_pass(x_hbm, src_hbm, dst_hbm, acc_out,
                src_v, dst_v, rows0, rows1, rows2, acc_sh, sem0, sem1, sem2):
        cid = lax.axis_index("c")
        sid = lax.axis_index("s")
        rows = (rows0, rows1, rows2)
        sems = (sem0, sem1, sem2)

        base = jnp.where(cid == 0, sid * CPT_FAST,
                         NS * CPT_FAST + sid * CPT_SLOW)
        n_gather = jnp.where(cid == 0, CPT_FAST // IDX_STAGE,
                             CPT_SLOW // IDX_STAGE)

        zrow = jnp.zeros((16,), jnp.float32)

        def fill_rows0(val):
            def body(i, carry):
                for j in range(D_FEAT // 16):
                    rows0[i, pl.ds(j * 16, 16)] = val
                return carry
            return body

        lax.fori_loop(0, CHUNK_G, fill_rows0(zrow), 0)

        r0 = sid * ROWS_PER_TILE
        for off, sz in _PARTS_G:
            pltpu.sync_copy(rows0.at[pl.ds(0, sz)],
                            acc_sh.at[pl.ds(r0 + off, sz)])

        plsc.subcore_barrier()

        def gather_body(s, carry):
            s8 = pl.multiple_of(base + s * IDX_STAGE, IDX_STAGE)
            pltpu.sync_copy(src_hbm.at[pl.ds(s8, IDX_STAGE)], src_v)
            pltpu.sync_copy(dst_hbm.at[pl.ds(s8, IDX_STAGE)], dst_v)
            handles = [None] * IDX_STAGE
            for j in range(2):
                handles[j] = pltpu.async_copy(
                    x_hbm.at[src_v.at[j]], rows[j], sems[j])
            for j in range(IDX_STAGE):
                if j + 2 < IDX_STAGE:
                    b = (j + 2) % 3
                    handles[j + 2] = pltpu.async_copy(
                        x_hbm.at[src_v.at[j + 2]], rows[b], sems[b])
                handles[j].wait()
                pltpu.sync_copy(rows[j % 3], acc_sh.at[dst_v.at[j]], add=True)
            return carry

        lax.fori_loop(0, n_gather, gather_body, 0)

        plsc.subcore_barrier()

        for off, sz in _PARTS_G:
            pltpu.sync_copy(acc_sh.at[pl.ds(r0 + off, sz)],
                            rows1.at[pl.ds(0, sz)])
            pltpu.sync_copy(rows1.at[pl.ds(0, sz)],
                            acc_out.at[cid, pl.ds(r0 + off, sz)])

    return sc_pass


@functools.lru_cache(maxsize=None)
def _make_deg_pass():
    @functools.partial(
        pl.kernel,
        mesh=_mesh(),
        out_type=[jax.ShapeDtypeStruct((NC, N_PAD, D_FEAT), jnp.float32)],
        scratch_types=[
            pltpu.VMEM((IDX_STAGE, CHUNK), jnp.int32),
            pltpu.VMEM((CHUNK, D_FEAT), jnp.float32),
            pltpu.VMEM_SHARED((N_PAD, D_FEAT), jnp.float32),
        ],
    )
    def deg_pass(dst_hbm, deg_out, dst_v, ones_v, deg_sh):
        cid = lax.axis_index("c")
        sid = lax.axis_index("s")
        wid = sid * NC + cid

        zrow = jnp.zeros((16,), jnp.float32)
        orow = jnp.full((16,), 1.0, jnp.float32)

        def fill_body(val):
            def body(i, carry):
                for j in range(D_FEAT // 16):
                    ones_v[i, pl.ds(j * 16, 16)] = val
                return carry
            return body

        lax.fori_loop(0, CHUNK, fill_body(zrow), 0)

        r0 = sid * ROWS_PER_TILE
        for off, sz in _PARTS:
            pltpu.sync_copy(ones_v.at[pl.ds(0, sz)],
                            deg_sh.at[pl.ds(r0 + off, sz)])

        lax.fori_loop(0, CHUNK, fill_body(orow), 0)

        plsc.subcore_barrier()

        def stage_body(s, carry):
            s8 = pl.multiple_of(s * IDX_STAGE, IDX_STAGE)
            pltpu.sync_copy(dst_hbm.at[wid, pl.ds(s8, IDX_STAGE)], dst_v)
            for j in range(IDX_STAGE):
                pltpu.sync_copy(ones_v, deg_sh.at[dst_v.at[j]], add=True)
            return carry

        lax.fori_loop(0, CPT // IDX_STAGE, stage_body, 0)

        plsc.subcore_barrier()

        for off, sz in _PARTS:
            pltpu.sync_copy(deg_sh.at[pl.ds(r0 + off, sz)],
                            ones_v.at[pl.ds(0, sz)])
            pltpu.sync_copy(ones_v.at[pl.ds(0, sz)],
                            deg_out.at[cid, pl.ds(r0 + off, sz)])

    return deg_pass


_TC_BLOCK = 400


def _tc_body(final, p_ref, deg_ref, x_ref, wl_ref, b_ref, wr_ref, o_ref):
    summed = p_ref[0] + p_ref[1]
    deg = deg_ref[0, :, 0:1] + deg_ref[1, :, 0:1]
    agg = summed / jnp.maximum(deg, 1.0)
    out = (jnp.dot(agg, wl_ref[...], precision=lax.Precision.HIGHEST)
           + b_ref[...]
           + jnp.dot(x_ref[...], wr_ref[...], precision=lax.Precision.HIGHEST))
    norm = jnp.sqrt(jnp.sum(out * out, axis=-1, keepdims=True))
    out = out / jnp.maximum(norm, 1e-12)
    out = jnp.maximum(out, 0.0)
    if final:
        m = jnp.max(out, axis=-1, keepdims=True)
        s = out - m
        lse = jnp.log(jnp.sum(jnp.exp(s), axis=-1, keepdims=True))
        out = s - lse
    o_ref[...] = out


def _tc_layer(p, degp, x_in, wl_t, b_row, wr_t, final, interpret=False):
    n = x_in.shape[0]
    return pl.pallas_call(
        functools.partial(_tc_body, final),
        grid=(n // _TC_BLOCK,),
        in_specs=[
            pl.BlockSpec((NC, _TC_BLOCK, D_FEAT), lambda i: (0, i, 0)),
            pl.BlockSpec((NC, _TC_BLOCK, D_FEAT), lambda i: (0, i, 0)),
            pl.BlockSpec((_TC_BLOCK, D_FEAT), lambda i: (i, 0)),
            pl.BlockSpec((D_FEAT, D_FEAT), lambda i: (0, 0)),
            pl.BlockSpec((1, D_FEAT), lambda i: (0, 0)),
            pl.BlockSpec((D_FEAT, D_FEAT), lambda i: (0, 0)),
        ],
        out_specs=pl.BlockSpec((_TC_BLOCK, D_FEAT), lambda i: (i, 0)),
        out_shape=jax.ShapeDtypeStruct((n, D_FEAT), jnp.float32),
        interpret=interpret,
    )(p, degp, x_in, wl_t, b_row, wr_t)


def kernel(x, edge_index, W_l1, b_l1, W_r1, W_l2, b_l2, W_r2):
    src = edge_index[0].astype(jnp.int32)
    dst = edge_index[1].astype(jnp.int32)
    pad = E_PAD - N_EDGES
    src_p = jnp.concatenate([src, jnp.zeros((pad,), jnp.int32)])
    dst_p = jnp.concatenate([dst, jnp.full((pad,), N_NODES, jnp.int32)])
    src_g = src_p.reshape(N_CHUNKS_G, CHUNK_G)
    dst_g = dst_p.reshape(N_CHUNKS_G, CHUNK_G)
    dst_d = dst_p.reshape(NW, CPT, CHUNK)

    d1 = _make_deg_pass()(dst_d)[0]
    p1 = _make_sc_pass()(x, src_g, dst_g)[0]
    h1 = _tc_layer(p1, d1, x, W_l1.T, b_l1.reshape(1, -1), W_r1.T, final=False)
    p2 = _make_sc_pass()(h1, src_g, dst_g)[0]
    out = _tc_layer(p2, d1, h1, W_l2.T, b_l2.reshape(1, -1), W_r2.T, final=True)
    return out


# speedup vs baseline: 1.0356x; 1.0003x over previous
import functools

import jax
import jax.numpy as jnp
from jax import lax
from jax.experimental import pallas as pl
from jax.experimental.pallas import tpu as pltpu
from jax.experimental.pallas import tpu_sc as plsc

N_NODES = 10000
D_FEAT = 128
N_EDGES = 320000

NC = 2
NS = 16
NW = NC * NS
CHUNK = 128
CPT = 80
E_PAD = NW * CPT * CHUNK
N_PAD = 10112
ROWS_PER_TILE = N_PAD // NS

_PARTS = ((0, 128), (128, 128), (256, 128), (384, 128), (512, 120))

IDX_STAGE = 8


def _mesh():
    return plsc.VectorSubcoreMesh(
        core_axis_name="c", subcore_axis_name="s", num_cores=NC)


CHUNK_G = 64
N_CHUNKS_G = E_PAD // CHUNK_G
CPT_FAST = 288
CPT_SLOW = (N_CHUNKS_G - NS * CPT_FAST) // NS
_PARTS_G = tuple((i * 64, 64) for i in range(9)) + ((576, 56),)


@functools.lru_cache(maxsize=None)
def _make_sc_pass():
    @functools.partial(
        pl.kernel,
        mesh=_mesh(),
        out_type=[jax.ShapeDtypeStruct((NC, N_PAD, D_FEAT), jnp.float32)],
        scratch_types=[
            pltpu.VMEM((IDX_STAGE, CHUNK_G), jnp.int32),
            pltpu.VMEM((IDX_STAGE, CHUNK_G), jnp.int32),
            pltpu.VMEM((CHUNK_G, D_FEAT), jnp.float32),
            pltpu.VMEM((CHUNK_G, D_FEAT), jnp.float32),
            pltpu.VMEM((CHUNK_G, D_FEAT), jnp.float32),
            pltpu.VMEM_SHARED((N_PAD, D_FEAT), jnp.float32),
            pltpu.SemaphoreType.DMA,
            pltpu.SemaphoreType.DMA,
            pltpu.SemaphoreType.DMA,
        ],
    )
    def sc_pass(x_hbm, src_hbm, dst_hbm, acc_out,
                src_v, dst_v, rows0, rows1, rows2, acc_sh, sem0, sem1, sem2):
        cid = lax.axis_index("c")
        sid = lax.axis_index("s")
        rows = (rows0, rows1, rows2)
        sems = (sem0, sem1, sem2)

        base = jnp.where(cid == 0, sid * CPT_FAST,
                         NS * CPT_FAST + sid * CPT_SLOW)
        n_gather = jnp.where(cid == 0, CPT_FAST // IDX_STAGE,
                             CPT_SLOW // IDX_STAGE)

        zrow = jnp.zeros((16,), jnp.float32)

        def fill_rows0(val):
            def body(i, carry):
                for j in range(D_FEAT // 16):
                    rows0[i, pl.ds(j * 16, 16)] = val
                return carry
            return body

        lax.fori_loop(0, CHUNK_G, fill_rows0(zrow), 0)

        r0 = sid * ROWS_PER_TILE
        for off, sz in _PARTS_G:
            pltpu.sync_copy(rows0.at[pl.ds(0, sz)],
                            acc_sh.at[pl.ds(r0 + off, sz)])

        plsc.subcore_barrier()

        def gather_body(s, carry):
            s8 = pl.multiple_of(base + s * IDX_STAGE, IDX_STAGE)
            pltpu.sync_copy(src_hbm.at[pl.ds(s8, IDX_STAGE)], src_v)
            pltpu.sync_copy(dst_hbm.at[pl.ds(s8, IDX_STAGE)], dst_v)
            handles = [None] * IDX_STAGE
            for j in range(2):
                handles[j] = pltpu.async_copy(
                    x_hbm.at[src_v.at[j]], rows[j], sems[j])
            for j in range(IDX_STAGE):
                if j + 2 < IDX_STAGE:
                    b = (j + 2) % 3
                    handles[j + 2] = pltpu.async_copy(
                        x_hbm.at[src_v.at[j + 2]], rows[b], sems[b])
                handles[j].wait()
                pltpu.sync_copy(rows[j % 3], acc_sh.at[dst_v.at[j]], add=True)
            return carry

        lax.fori_loop(0, n_gather, gather_body, 0)

        plsc.subcore_barrier()

        for off, sz in _PARTS_G:
            pltpu.sync_copy(acc_sh.at[pl.ds(r0 + off, sz)],
                            rows1.at[pl.ds(0, sz)])
            pltpu.sync_copy(rows1.at[pl.ds(0, sz)],
                            acc_out.at[cid, pl.ds(r0 + off, sz)])

    return sc_pass


@functools.lru_cache(maxsize=None)
def _make_deg_pass():
    @functools.partial(
        pl.kernel,
        mesh=_mesh(),
        out_type=[jax.ShapeDtypeStruct((NC, N_PAD, D_FEAT), jnp.float32)],
        scratch_types=[
            pltpu.VMEM((IDX_STAGE, CHUNK), jnp.int32),
            pltpu.VMEM((CHUNK, D_FEAT), jnp.float32),
            pltpu.VMEM_SHARED((N_PAD, D_FEAT), jnp.float32),
        ],
    )
    def deg_pass(dst_hbm, deg_out, dst_v, ones_v, deg_sh):
        cid = lax.axis_index("c")
        sid = lax.axis_index("s")
        wid = sid * NC + cid

        zrow = jnp.zeros((16,), jnp.float32)
        orow = jnp.full((16,), 1.0, jnp.float32)

        def fill_body(val):
            def body(i, carry):
                for j in range(D_FEAT // 16):
                    ones_v[i, pl.ds(j * 16, 16)] = val
                return carry
            return body

        lax.fori_loop(0, CHUNK, fill_body(zrow), 0)

        r0 = sid * ROWS_PER_TILE
        for off, sz in _PARTS:
            pltpu.sync_copy(ones_v.at[pl.ds(0, sz)],
                            deg_sh.at[pl.ds(r0 + off, sz)])

        lax.fori_loop(0, CHUNK, fill_body(orow), 0)

        plsc.subcore_barrier()

        def stage_body(s, carry):
            s8 = pl.multiple_of(s * IDX_STAGE, IDX_STAGE)
            pltpu.sync_copy(dst_hbm.at[wid, pl.ds(s8, IDX_STAGE)], dst_v)
            for j in range(IDX_STAGE):
                pltpu.sync_copy(ones_v, deg_sh.at[dst_v.at[j]], add=True)
            return carry

        lax.fori_loop(0, CPT // IDX_STAGE, stage_body, 0)

        plsc.subcore_barrier()

        for off, sz in _PARTS:
            pltpu.sync_copy(deg_sh.at[pl.ds(r0 + off, sz)],
                            ones_v.at[pl.ds(0, sz)])
            pltpu.sync_copy(ones_v.at[pl.ds(0, sz)],
                            deg_out.at[cid, pl.ds(r0 + off, sz)])

    return deg_pass


_TC_BLOCK = 400


def _tc_body(final, p_ref, deg_ref, x_ref, wl_ref, b_ref, wr_ref, o_ref):
    summed = p_ref[0] + p_ref[1]
    deg = deg_ref[0, :, 0:1] + deg_ref[1, :, 0:1]
    agg = summed / jnp.maximum(deg, 1.0)
    out = (jnp.dot(agg, wl_ref[...], precision=lax.Precision.HIGHEST)
           + b_ref[...]
           + jnp.dot(x_ref[...], wr_ref[...], precision=lax.Precision.HIGHEST))
    norm = jnp.sqrt(jnp.sum(out * out, axis=-1, keepdims=True))
    out = out / jnp.maximum(norm, 1e-12)
    out = jnp.maximum(out, 0.0)
    if final:
        m = jnp.max(out, axis=-1, keepdims=True)
        s = out - m
        lse = jnp.log(jnp.sum(jnp.exp(s), axis=-1, keepdims=True))
        out = s - lse
    o_ref[...] = out


def _tc_layer(p, degp, x_in, wl_t, b_row, wr_t, final):
    n = x_in.shape[0]
    return pl.pallas_call(
        functools.partial(_tc_body, final),
        grid=(n // _TC_BLOCK,),
        in_specs=[
            pl.BlockSpec((NC, _TC_BLOCK, D_FEAT), lambda i: (0, i, 0)),
            pl.BlockSpec((NC, _TC_BLOCK, D_FEAT), lambda i: (0, i, 0)),
            pl.BlockSpec((_TC_BLOCK, D_FEAT), lambda i: (i, 0)),
            pl.BlockSpec((D_FEAT, D_FEAT), lambda i: (0, 0)),
            pl.BlockSpec((1, D_FEAT), lambda i: (0, 0)),
            pl.BlockSpec((D_FEAT, D_FEAT), lambda i: (0, 0)),
        ],
        out_specs=pl.BlockSpec((_TC_BLOCK, D_FEAT), lambda i: (i, 0)),
        out_shape=jax.ShapeDtypeStruct((n, D_FEAT), jnp.float32),
    )(p, degp, x_in, wl_t, b_row, wr_t)


def kernel(x, edge_index, W_l1, b_l1, W_r1, W_l2, b_l2, W_r2):
    src = edge_index[0].astype(jnp.int32)
    dst = edge_index[1].astype(jnp.int32)
    pad = E_PAD - N_EDGES
    src_p = jnp.concatenate([src, jnp.zeros((pad,), jnp.int32)])
    dst_p = jnp.concatenate([dst, jnp.full((pad,), N_NODES, jnp.int32)])
    src_g = src_p.reshape(N_CHUNKS_G, CHUNK_G)
    dst_g = dst_p.reshape(N_CHUNKS_G, CHUNK_G)
    dst_d = dst_p.reshape(NW, CPT, CHUNK)

    d1 = _make_deg_pass()(dst_d)[0]
    p1 = _make_sc_pass()(x, src_g, dst_g)[0]
    h1 = _tc_layer(p1, d1, x, W_l1.T, b_l1.reshape(1, -1), W_r1.T, final=False)
    p2 = _make_sc_pass()(h1, src_g, dst_g)[0]
    out = _tc_layer(p2, d1, h1, W_l2.T, b_l2.reshape(1, -1), W_r2.T, final=True)
    return out
